# baseline (speedup 1.0000x reference)
import functools

import numpy as np
import jax
import jax.numpy as jnp
from jax import lax
from jax.experimental import pallas as pl
from jax.experimental.pallas import tpu as pltpu
from jax.experimental.pallas import tpu_sc as plsc

_BINS = (64, 256, 64, 256, 512, 256, 512, 512, 256)
_OFFS = tuple(int(x) for x in np.concatenate([[0], np.cumsum(_BINS)[:-1]]))
_NT = len(_BINS)
_EMB = 8
_B = 16384

_NW = 32
_BPW = _B // _NW
_CHUNK = 128
_NC = _BPW // _CHUNK

_mesh = plsc.VectorSubcoreMesh(core_axis_name="c", subcore_axis_name="s")


@functools.partial(
    pl.kernel,
    mesh=_mesh,
    compiler_params=pltpu.CompilerParams(
        use_tc_tiling_on_sc=False, needs_layout_passes=False),
    out_type=jax.ShapeDtypeStruct((_B, _NT * _EMB), jnp.float32),
    scratch_types=[
        pltpu.VMEM((_BPW, _NT), jnp.int32),
        pltpu.VMEM((_NT * _NC, _CHUNK), jnp.int32),
        [pltpu.VMEM((_BPW, _EMB), jnp.float32) for _ in range(_NT)],
        pltpu.SemaphoreType.DMA,
        pltpu.SemaphoreType.DMA,
    ],
)
def _emb_lookup(cate_hbm, table_hbm, out_hbm,
                raw_v, idx_v, rows_vs, gsem, osem):
    wid = lax.axis_index("s") * 2 + lax.axis_index("c")
    row0 = wid * _BPW
    pltpu.sync_copy(cate_hbm.at[pl.ds(row0, _BPW)], raw_v)

    base_iota = lax.iota(jnp.int32, 16)
    gather_cps = []
    for i in range(_NT):
        col = jnp.full((16,), i, jnp.int32)
        for c in range(_NC):
            for l in range(_CHUNK // 16):
                r = c * _CHUNK + l * 16
                raw16 = plsc.load_gather(raw_v, [base_iota + r, col])
                idx_v[i * _NC + c, pl.ds(l * 16, 16)] = \
                    (raw16 & (_BINS[i] - 1)) + _OFFS[i]
            gather_cps.append(pltpu.async_copy(
                table_hbm.at[idx_v.at[i * _NC + c]],
                rows_vs[i].at[pl.ds(c * _CHUNK, _CHUNK)],
                gsem,
            ))
    for cp in gather_cps:
        cp.wait()
    out_cps = []
    for i in range(_NT):
        out_cps.append(pltpu.async_copy(
            rows_vs[i],
            out_hbm.at[pl.ds(row0, _BPW), pl.ds(i * _EMB, _EMB)],
            osem,
        ))
    for cp in out_cps:
        cp.wait()


def kernel(num_feats, cate_feats, W0, W1, W2, W3, W4, W5, W6, W7, W8):
    del num_feats
    flat_table = jnp.concatenate(
        [W0, W1, W2, W3, W4, W5, W6, W7, W8], axis=0)
    return _emb_lookup(cate_feats, flat_table)


# speedup vs baseline: 1.3857x; 1.3857x over previous
import functools

import numpy as np
import jax
import jax.numpy as jnp
from jax import lax
from jax.experimental import pallas as pl
from jax.experimental.pallas import tpu as pltpu
from jax.experimental.pallas import tpu_sc as plsc

_BINS = (64, 256, 64, 256, 512, 256, 512, 512, 256)
_OFFS = tuple(int(x) for x in np.concatenate([[0], np.cumsum(_BINS)[:-1]]))
_NT = len(_BINS)
_EMB = 8
_B = 16384

_NW = 32
_BPW = _B // _NW
_CHUNK = 128
_NC = _BPW // _CHUNK
_NCH = _NT * _NC

_mesh = plsc.VectorSubcoreMesh(core_axis_name="c", subcore_axis_name="s")


@functools.partial(
    pl.kernel,
    mesh=_mesh,
    compiler_params=pltpu.CompilerParams(
        use_tc_tiling_on_sc=False, needs_layout_passes=False),
    out_type=jax.ShapeDtypeStruct((_NT, _B // _CHUNK, _EMB, _CHUNK),
                                  jnp.float32),
    scratch_types=[
        pltpu.VMEM((_NT, _BPW), jnp.int32),
        pltpu.VMEM((_NCH, _CHUNK), jnp.int32),
        pltpu.VMEM((_NCH * _CHUNK, _EMB), jnp.float32),
        pltpu.VMEM((_EMB, _CHUNK), jnp.float32),
        pltpu.SemaphoreType.DMA,
        pltpu.SemaphoreType.DMA,
    ],
)
def _emb_lookup(cate_hbm, table_hbm, out_hbm,
                raw_v, idx_v, rows_v, tile_v, gsem, osem):
    wid = lax.axis_index("s") * 2 + lax.axis_index("c")
    b0 = wid * _BPW

    in_cps = [
        pltpu.async_copy(cate_hbm.at[i, pl.ds(b0, _BPW)], raw_v.at[i], gsem)
        for i in range(_NT)
    ]
    for cp in in_cps:
        cp.wait()

    cps = []
    for i in range(_NT):
        for c in range(_NC):
            j = i * _NC + c
            for l in range(_CHUNK // 16):
                raw16 = raw_v[i, pl.ds(c * _CHUNK + l * 16, 16)]
                idx_v[j, pl.ds(l * 16, 16)] = \
                    (raw16 & (_BINS[i] - 1)) + _OFFS[i]
            cps.append(pltpu.async_copy(
                table_hbm.at[idx_v.at[j]],
                rows_v.at[pl.ds(j * _CHUNK, _CHUNK)],
                gsem,
            ))

    iota = lax.iota(jnp.int32, 16)

    def drain(j, carry):
        pltpu.make_async_copy(
            table_hbm.at[idx_v.at[j]],
            rows_v.at[pl.ds(j * _CHUNK, _CHUNK)],
            gsem,
        ).wait()
        base = j * _CHUNK
        for l in range(_CHUNK // 16):
            rvec = iota + (base + l * 16)
            for e in range(_EMB):
                tile_v[e, pl.ds(l * 16, 16)] = plsc.load_gather(
                    rows_v, [rvec, jnp.full((16,), e, jnp.int32)])
        i = j // _NC
        tc = _NC * wid + (j - i * _NC)
        pltpu.sync_copy(tile_v, out_hbm.at[i, tc])
        return carry

    lax.fori_loop(0, _NCH, drain, 0)


def kernel(num_feats, cate_feats, W0, W1, W2, W3, W4, W5, W6, W7, W8):
    del num_feats
    flat_table = jnp.concatenate(
        [W0, W1, W2, W3, W4, W5, W6, W7, W8], axis=0)
    x = _emb_lookup(cate_feats.T, flat_table)
    return x.transpose(1, 3, 0, 2).reshape(_B, _NT * _EMB)


# speedup vs baseline: 1.4392x; 1.0386x over previous
import functools

import numpy as np
import jax
import jax.numpy as jnp
from jax import lax
from jax.experimental import pallas as pl
from jax.experimental.pallas import tpu as pltpu
from jax.experimental.pallas import tpu_sc as plsc

_BINS = (64, 256, 64, 256, 512, 256, 512, 512, 256)
_OFFS = tuple(int(x) for x in np.concatenate([[0], np.cumsum(_BINS)[:-1]]))
_NT = len(_BINS)
_EMB = 8
_B = 16384

_NW = 32
_BPW = _B // _NW
_CHUNK = 128
_NC = _BPW // _CHUNK
_NCH = _NT * _NC

_mesh = plsc.VectorSubcoreMesh(core_axis_name="c", subcore_axis_name="s")


@functools.partial(
    pl.kernel,
    mesh=_mesh,
    compiler_params=pltpu.CompilerParams(
        use_tc_tiling_on_sc=False, needs_layout_passes=False),
    out_type=jax.ShapeDtypeStruct((_NT, _B // _CHUNK, _EMB, _CHUNK),
                                  jnp.float32),
    scratch_types=[
        pltpu.VMEM((_NT, _BPW), jnp.int32),
        pltpu.VMEM((_NCH, _CHUNK), jnp.int32),
        pltpu.VMEM((_NCH * _CHUNK, _EMB), jnp.float32),
        pltpu.VMEM((4 * _EMB, _CHUNK), jnp.float32),
        pltpu.SemaphoreType.DMA,
        pltpu.SemaphoreType.DMA,
    ],
)
def _emb_lookup(cate_hbm, table_hbm, out_hbm,
                raw_v, idx_v, rows_v, tile_v, gsem, osem):
    wid = lax.axis_index("s") * 2 + lax.axis_index("c")
    b0 = wid * _BPW

    in_cps = [
        pltpu.async_copy(cate_hbm.at[i, pl.ds(b0, _BPW)], raw_v.at[i], gsem)
        for i in range(_NT)
    ]
    for cp in in_cps:
        cp.wait()

    cps = []
    for i in range(_NT):
        for c in range(_NC):
            j = i * _NC + c
            for l in range(_CHUNK // 16):
                raw16 = raw_v[i, pl.ds(c * _CHUNK + l * 16, 16)]
                idx_v[j, pl.ds(l * 16, 16)] = \
                    (raw16 & (_BINS[i] - 1)) + _OFFS[i]
            cps.append(pltpu.async_copy(
                table_hbm.at[idx_v.at[j]],
                rows_v.at[pl.ds(j * _CHUNK, _CHUNK)],
                gsem,
            ))

    iota = lax.iota(jnp.int32, 16)

    def drain(j, carry):
        pltpu.make_async_copy(
            table_hbm.at[idx_v.at[j]],
            rows_v.at[pl.ds(j * _CHUNK, _CHUNK)],
            gsem,
        ).wait()
        i = j // _NC
        tc = _NC * wid + (j - i * _NC)
        slot = (j % 4) * _EMB
        @pl.when(j >= 4)
        def _():
            pltpu.make_async_copy(
                tile_v.at[pl.ds(slot, _EMB)], out_hbm.at[i, tc], osem
            ).wait()
        base = j * _CHUNK
        for l in range(_CHUNK // 16):
            rvec = iota + (base + l * 16)
            for e in range(_EMB):
                tile_v[slot + e, pl.ds(l * 16, 16)] = plsc.load_gather(
                    rows_v, [rvec, jnp.full((16,), e, jnp.int32)])
        pltpu.async_copy(
            tile_v.at[pl.ds(slot, _EMB)], out_hbm.at[i, tc], osem)
        return carry

    lax.fori_loop(0, _NCH, drain, 0)
    for _ in range(4):
        pltpu.make_async_copy(
            tile_v.at[pl.ds(0, _EMB)], out_hbm.at[0, 0], osem
        ).wait()


def kernel(num_feats, cate_feats, W0, W1, W2, W3, W4, W5, W6, W7, W8):
    del num_feats
    flat_table = jnp.concatenate(
        [W0, W1, W2, W3, W4, W5, W6, W7, W8], axis=0)
    x = _emb_lookup(cate_feats.T, flat_table)
    return x.transpose(1, 3, 0, 2).reshape(_B, _NT * _EMB)


# speedup vs baseline: 1.7181x; 1.1938x over previous
import functools

import numpy as np
import jax
import jax.numpy as jnp
from jax import lax
from jax.experimental import pallas as pl
from jax.experimental.pallas import tpu as pltpu
from jax.experimental.pallas import tpu_sc as plsc

_BINS = (64, 256, 64, 256, 512, 256, 512, 512, 256)
_OFFS = tuple(int(x) for x in np.concatenate([[0], np.cumsum(_BINS)[:-1]]))
_NT = len(_BINS)
_EMB = 8
_B = 16384

_NW = 32
_BPW = _B // _NW
_CHUNK = 128
_NC = _BPW // _CHUNK
_NCH = _NT * _NC

_mesh = plsc.VectorSubcoreMesh(core_axis_name="c", subcore_axis_name="s")


@functools.partial(
    pl.kernel,
    mesh=_mesh,
    compiler_params=pltpu.CompilerParams(
        use_tc_tiling_on_sc=False, needs_layout_passes=False),
    out_type=jax.ShapeDtypeStruct((_NT, _B // _CHUNK, _EMB, _CHUNK),
                                  jnp.float32),
    scratch_types=[
        pltpu.VMEM((_NT, _BPW), jnp.int32),
        pltpu.VMEM((_NCH, _CHUNK), jnp.int32),
        pltpu.VMEM((_NCH * _CHUNK, _EMB), jnp.float32),
        pltpu.VMEM((4 * _EMB, _CHUNK), jnp.float32),
        pltpu.SemaphoreType.DMA,
        pltpu.SemaphoreType.DMA,
    ],
)
def _emb_lookup(cate_hbm, table_hbm, out_hbm,
                raw_v, idx_v, rows_v, tile_v, gsem, osem):
    wid = lax.axis_index("s") * 2 + lax.axis_index("c")
    b0 = wid * _BPW

    in_cps = [
        pltpu.async_copy(cate_hbm.at[i, pl.ds(b0, _BPW)], raw_v.at[i], gsem)
        for i in range(_NT)
    ]
    for cp in in_cps:
        cp.wait()

    cps = []
    for i in range(_NT):
        for c in range(_NC):
            j = i * _NC + c
            for l in range(_CHUNK // 16):
                raw16 = raw_v[i, pl.ds(c * _CHUNK + l * 16, 16)]
                idx_v[j, pl.ds(l * 16, 16)] = \
                    (raw16 & (_BINS[i] - 1)) + _OFFS[i]
            cps.append(pltpu.async_copy(
                table_hbm.at[idx_v.at[j]],
                rows_v.at[pl.ds(j * _CHUNK, _CHUNK)],
                gsem,
            ))

    iota = lax.iota(jnp.int32, 16)

    for cp in cps:
        cp.wait()


def kernel(num_feats, cate_feats, W0, W1, W2, W3, W4, W5, W6, W7, W8):
    del num_feats
    flat_table = jnp.concatenate(
        [W0, W1, W2, W3, W4, W5, W6, W7, W8], axis=0)
    x = _emb_lookup(cate_feats.T, flat_table)
    return x.transpose(1, 3, 0, 2).reshape(_B, _NT * _EMB)


# speedup vs baseline: 2.1043x; 1.2248x over previous
import functools

import numpy as np
import jax
import jax.numpy as jnp
from jax import lax
from jax.experimental import pallas as pl
from jax.experimental.pallas import tpu as pltpu
from jax.experimental.pallas import tpu_sc as plsc

_BINS = (64, 256, 64, 256, 512, 256, 512, 512, 256)
_NT = len(_BINS)
_EMB = 8
_PADW = _EMB + 1
_B = 16384
_ROWS = int(np.sum(_BINS))

_NW = 32
_BPW = _B // _NW
_CHUNK = 128
_NC = _BPW // _CHUNK
_NCH = _NT * _NC

_MASK16 = np.zeros(16, np.int32)
_MASK16[:_NT] = np.array([b - 1 for b in _BINS], np.int32)
_OFF16 = np.zeros(16, np.int32)
_OFF16[:_NT] = np.concatenate([[0], np.cumsum(_BINS)[:-1]]).astype(np.int32)

_mesh = plsc.VectorSubcoreMesh(core_axis_name="c", subcore_axis_name="s")


@functools.partial(
    pl.kernel,
    mesh=_mesh,
    compiler_params=pltpu.CompilerParams(
        use_tc_tiling_on_sc=False, needs_layout_passes=False),
    out_type=jax.ShapeDtypeStruct((_NT, _B // _CHUNK, _EMB, _CHUNK),
                                  jnp.float32),
    scratch_types=[
        pltpu.VMEM((_ROWS, _PADW), jnp.float32),
        pltpu.VMEM((_NT, _BPW), jnp.int32),
        pltpu.VMEM((16,), jnp.int32),
        pltpu.VMEM((16,), jnp.int32),
        pltpu.VMEM((4 * _EMB, _CHUNK), jnp.float32),
        pltpu.SemaphoreType.DMA,
        pltpu.SemaphoreType.DMA,
    ],
)
def _emb_lookup(cate_hbm, table_hbm, mask_hbm, off_hbm, out_hbm,
                table_v, raw_v, mask_v, off_v, tile_v, gsem, osem):
    wid = lax.axis_index("s") * 2 + lax.axis_index("c")
    b0 = wid * _BPW

    in_cps = [
        pltpu.async_copy(table_hbm, table_v, gsem),
        pltpu.async_copy(mask_hbm, mask_v, gsem),
        pltpu.async_copy(off_hbm, off_v, gsem),
    ]
    in_cps += [
        pltpu.async_copy(cate_hbm.at[i, pl.ds(b0, _BPW)], raw_v.at[i], gsem)
        for i in range(_NT)
    ]
    for cp in in_cps:
        cp.wait()

    efulls = [jnp.full((16,), e, jnp.int32) for e in range(_EMB)]

    def tile_body(j, carry):
        i = j // _NC
        c = j - i * _NC
        tc = _NC * wid + c
        ivec = jnp.full((16,), 1, jnp.int32) * i
        maskv = plsc.load_gather(mask_v, [ivec])
        offv = plsc.load_gather(off_v, [ivec])
        slot = (j % 4) * _EMB
        @pl.when(j >= 4)
        def _():
            pltpu.make_async_copy(
                tile_v.at[pl.ds(slot, _EMB)], out_hbm.at[i, tc], osem
            ).wait()
        for l in range(_CHUNK // 16):
            raw16 = raw_v[i, pl.ds(c * _CHUNK + l * 16, 16)]
            idx16 = (raw16 & maskv) + offv
            for e in range(_EMB):
                tile_v[slot + e, pl.ds(l * 16, 16)] = plsc.load_gather(
                    table_v, [idx16, efulls[e]])
        pltpu.async_copy(
            tile_v.at[pl.ds(slot, _EMB)], out_hbm.at[i, tc], osem)
        return carry

    lax.fori_loop(0, _NCH, tile_body, 0)
    for _ in range(4):
        pltpu.make_async_copy(
            tile_v.at[pl.ds(0, _EMB)], out_hbm.at[0, 0], osem
        ).wait()


def kernel(num_feats, cate_feats, W0, W1, W2, W3, W4, W5, W6, W7, W8):
    del num_feats
    flat_table = jnp.pad(
        jnp.concatenate([W0, W1, W2, W3, W4, W5, W6, W7, W8], axis=0),
        ((0, 0), (0, _PADW - _EMB)))
    x = _emb_lookup(cate_feats.T, flat_table,
                    jnp.asarray(_MASK16), jnp.asarray(_OFF16))
    return x.transpose(1, 3, 0, 2).reshape(_B, _NT * _EMB)


# speedup vs baseline: 2.6472x; 1.2580x over previous
import functools

import numpy as np
import jax
import jax.numpy as jnp
from jax import lax
from jax.experimental import pallas as pl
from jax.experimental.pallas import tpu as pltpu
from jax.experimental.pallas import tpu_sc as plsc

_BINS = (64, 256, 64, 256, 512, 256, 512, 512, 256)
_NT = len(_BINS)
_EMB = 8
_PADW = _EMB + 1
_B = 16384
_ROWS = int(np.sum(_BINS))

_NW = 32
_BPW = _B // _NW
_CHUNK = 128
_NC = _BPW // _CHUNK
_NCH = _NT * _NC

_MASK16 = np.zeros(16, np.int32)
_MASK16[:_NT] = np.array([b - 1 for b in _BINS], np.int32)
_OFF16 = np.zeros(16, np.int32)
_OFF16[:_NT] = np.concatenate([[0], np.cumsum(_BINS)[:-1]]).astype(np.int32)

_mesh = plsc.VectorSubcoreMesh(core_axis_name="c", subcore_axis_name="s")


@functools.partial(
    pl.kernel,
    mesh=_mesh,
    compiler_params=pltpu.CompilerParams(
        use_tc_tiling_on_sc=False, needs_layout_passes=False),
    out_type=jax.ShapeDtypeStruct((_NT, _B // _CHUNK, _EMB, _CHUNK),
                                  jnp.float32),
    scratch_types=[
        pltpu.VMEM((_ROWS * _PADW,), jnp.float32),
        pltpu.VMEM((_NT, _BPW), jnp.int32),
        pltpu.VMEM((16,), jnp.int32),
        pltpu.VMEM((16,), jnp.int32),
        pltpu.VMEM((4 * _EMB, _CHUNK), jnp.float32),
        pltpu.SemaphoreType.DMA,
        pltpu.SemaphoreType.DMA,
    ],
)
def _emb_lookup(cate_hbm, table_hbm, mask_hbm, off_hbm, out_hbm,
                table_v, raw_v, mask_v, off_v, tile_v, gsem, osem):
    wid = lax.axis_index("s") * 2 + lax.axis_index("c")
    b0 = wid * _BPW

    in_cps = [
        pltpu.async_copy(table_hbm, table_v, gsem),
        pltpu.async_copy(mask_hbm, mask_v, gsem),
        pltpu.async_copy(off_hbm, off_v, gsem),
    ]
    in_cps += [
        pltpu.async_copy(cate_hbm.at[i, pl.ds(b0, _BPW)], raw_v.at[i], gsem)
        for i in range(_NT)
    ]
    for cp in in_cps:
        cp.wait()

    efulls = [jnp.full((16,), e, jnp.int32) for e in range(_EMB)]

    def tile_body(j, carry):
        i = j // _NC
        c = j - i * _NC
        tc = _NC * wid + c
        ivec = jnp.full((16,), 1, jnp.int32) * i
        maskv = plsc.load_gather(mask_v, [ivec])
        offv = plsc.load_gather(off_v, [ivec])
        slot = (j % 4) * _EMB
        @pl.when(j >= 4)
        def _():
            pltpu.make_async_copy(
                tile_v.at[pl.ds(slot, _EMB)], out_hbm.at[i, tc], osem
            ).wait()
        for l in range(_CHUNK // 16):
            raw16 = raw_v[i, pl.ds(c * _CHUNK + l * 16, 16)]
            idx16 = (raw16 & maskv) + offv
            base16 = idx16 * _PADW
            for e in range(_EMB):
                tile_v[slot + e, pl.ds(l * 16, 16)] = plsc.load_gather(
                    table_v, [base16 + e])
        pltpu.async_copy(
            tile_v.at[pl.ds(slot, _EMB)], out_hbm.at[i, tc], osem)
        return carry

    lax.fori_loop(0, _NCH, tile_body, 0)
    for _ in range(4):
        pltpu.make_async_copy(
            tile_v.at[pl.ds(0, _EMB)], out_hbm.at[0, 0], osem
        ).wait()


def kernel(num_feats, cate_feats, W0, W1, W2, W3, W4, W5, W6, W7, W8):
    del num_feats
    flat_table = jnp.pad(
        jnp.concatenate([W0, W1, W2, W3, W4, W5, W6, W7, W8], axis=0),
        ((0, 0), (0, _PADW - _EMB))).reshape(_ROWS * _PADW)
    x = _emb_lookup(cate_feats.T, flat_table,
                    jnp.asarray(_MASK16), jnp.asarray(_OFF16))
    return x.transpose(1, 3, 0, 2).reshape(_B, _NT * _EMB)


# speedup vs baseline: 3.2388x; 1.2235x over previous
import functools

import numpy as np
import jax
import jax.numpy as jnp
from jax import lax
from jax.experimental import pallas as pl
from jax.experimental.pallas import tpu as pltpu
from jax.experimental.pallas import tpu_sc as plsc

_BINS = (64, 256, 64, 256, 512, 256, 512, 512, 256)
_NT = len(_BINS)
_EMB = 8
_PADW = _EMB + 1
_B = 16384
_ROWS = int(np.sum(_BINS))

_NW = 32
_BPW = _B // _NW
_CHUNK = 128
_NC = _BPW // _CHUNK
_NCH = _NT * _NC

_MASK16 = np.zeros(16, np.int32)
_MASK16[:_NT] = np.array([b - 1 for b in _BINS], np.int32)
_OFF16 = np.zeros(16, np.int32)
_OFF16[:_NT] = np.concatenate([[0], np.cumsum(_BINS)[:-1]]).astype(np.int32)

_mesh = plsc.VectorSubcoreMesh(core_axis_name="c", subcore_axis_name="s")


@functools.partial(
    pl.kernel,
    mesh=_mesh,
    compiler_params=pltpu.CompilerParams(
        use_tc_tiling_on_sc=False, needs_layout_passes=False),
    out_type=jax.ShapeDtypeStruct((_NT, _B // _CHUNK, _EMB, _CHUNK),
                                  jnp.float32),
    scratch_types=[
        pltpu.VMEM((_ROWS * _PADW,), jnp.float32),
        pltpu.VMEM((_NT, _BPW), jnp.int32),
        pltpu.VMEM((16,), jnp.int32),
        pltpu.VMEM((16,), jnp.int32),
        pltpu.VMEM((4 * _EMB, _CHUNK), jnp.float32),
        pltpu.SemaphoreType.DMA,
        pltpu.SemaphoreType.DMA,
    ],
)
def _emb_lookup(cate_hbm, table_hbm, mask_hbm, off_hbm, out_hbm,
                table_v, raw_v, mask_v, off_v, tile_v, gsem, osem):
    wid = lax.axis_index("s") * 2 + lax.axis_index("c")
    b0 = wid * _BPW

    in_cps = [
        pltpu.async_copy(table_hbm, table_v, gsem),
        pltpu.async_copy(mask_hbm, mask_v, gsem),
        pltpu.async_copy(off_hbm, off_v, gsem),
    ]
    in_cps += [
        pltpu.async_copy(cate_hbm.at[i, pl.ds(b0, _BPW)], raw_v.at[i], gsem)
        for i in range(_NT)
    ]
    for cp in in_cps:
        cp.wait()

    efulls = [jnp.full((16,), e, jnp.int32) for e in range(_EMB)]

    def tile_body(j, carry):
        i = j // _NC
        c = j - i * _NC
        tc = _NC * wid + c
        ivec = jnp.full((16,), 1, jnp.int32) * i
        maskv = plsc.load_gather(mask_v, [ivec])
        offv = plsc.load_gather(off_v, [ivec])
        slot = (j % 4) * _EMB
        @pl.when(j >= 4)
        def _():
            pltpu.make_async_copy(
                tile_v.at[pl.ds(slot, _EMB)], out_hbm.at[i, tc], osem
            ).wait()
        for g in range(2):
            ls = [4 * g + k for k in range(4)]
            bases = []
            for l in ls:
                raw16 = raw_v[i, pl.ds(c * _CHUNK + l * 16, 16)]
                idx16 = (raw16 & maskv) + offv
                bases.append(idx16 * _PADW)
            for e in range(_EMB):
                vals = [plsc.load_gather(table_v, [b + e]) for b in bases]
                for l, v in zip(ls, vals):
                    tile_v[slot + e, pl.ds(l * 16, 16)] = v
        pltpu.async_copy(
            tile_v.at[pl.ds(slot, _EMB)], out_hbm.at[i, tc], osem)
        return carry

    lax.fori_loop(0, _NCH, tile_body, 0)
    for _ in range(4):
        pltpu.make_async_copy(
            tile_v.at[pl.ds(0, _EMB)], out_hbm.at[0, 0], osem
        ).wait()


def kernel(num_feats, cate_feats, W0, W1, W2, W3, W4, W5, W6, W7, W8):
    del num_feats
    flat_table = jnp.pad(
        jnp.concatenate([W0, W1, W2, W3, W4, W5, W6, W7, W8], axis=0),
        ((0, 0), (0, _PADW - _EMB))).reshape(_ROWS * _PADW)
    x = _emb_lookup(cate_feats.T, flat_table,
                    jnp.asarray(_MASK16), jnp.asarray(_OFF16))
    return x.transpose(1, 3, 0, 2).reshape(_B, _NT * _EMB)


# speedup vs baseline: 3.2580x; 1.0059x over previous
import functools

import numpy as np
import jax
import jax.numpy as jnp
from jax import lax
from jax.experimental import pallas as pl
from jax.experimental.pallas import tpu as pltpu
from jax.experimental.pallas import tpu_sc as plsc

_BINS = (64, 256, 64, 256, 512, 256, 512, 512, 256)
_NT = len(_BINS)
_EMB = 8
_PADW = _EMB + 1
_B = 16384
_ROWS = int(np.sum(_BINS))

_NW = 32
_BPW = _B // _NW
_CHUNK = 128
_NC = _BPW // _CHUNK
_NCH = _NT * _NC

_MASK16 = np.zeros(16, np.int32)
_MASK16[:_NT] = np.array([b - 1 for b in _BINS], np.int32)
_OFF16 = np.zeros(16, np.int32)
_OFF16[:_NT] = np.concatenate([[0], np.cumsum(_BINS)[:-1]]).astype(np.int32)

_mesh = plsc.VectorSubcoreMesh(core_axis_name="c", subcore_axis_name="s")


@functools.partial(
    pl.kernel,
    mesh=_mesh,
    compiler_params=pltpu.CompilerParams(
        use_tc_tiling_on_sc=False, needs_layout_passes=False),
    out_type=jax.ShapeDtypeStruct((_NT, _B // _CHUNK, _EMB, _CHUNK),
                                  jnp.float32),
    scratch_types=[
        pltpu.VMEM((_ROWS * _PADW,), jnp.float32),
        pltpu.VMEM((_NT, _BPW), jnp.int32),
        pltpu.VMEM((16,), jnp.int32),
        pltpu.VMEM((16,), jnp.int32),
        pltpu.VMEM((_NCH * _EMB, 16), jnp.int32),
        pltpu.VMEM((4 * _EMB, _CHUNK), jnp.float32),
        pltpu.SemaphoreType.DMA,
        pltpu.SemaphoreType.DMA,
        pltpu.SemaphoreType.DMA,
    ],
)
def _emb_lookup(cate_hbm, table_hbm, mask_hbm, off_hbm, out_hbm,
                table_v, raw_v, mask_v, off_v, base_v, tile_v,
                gsem, osem, tsem):
    wid = lax.axis_index("s") * 2 + lax.axis_index("c")
    b0 = wid * _BPW

    table_cp = pltpu.async_copy(table_hbm, table_v, tsem)
    in_cps = [
        pltpu.async_copy(mask_hbm, mask_v, gsem),
        pltpu.async_copy(off_hbm, off_v, gsem),
    ]
    in_cps += [
        pltpu.async_copy(cate_hbm.at[i, pl.ds(b0, _BPW)], raw_v.at[i], gsem)
        for i in range(_NT)
    ]
    for cp in in_cps:
        cp.wait()

    def idx_body(j, carry):
        i = j // _NC
        c = j - i * _NC
        ivec = jnp.full((16,), 1, jnp.int32) * i
        maskv = plsc.load_gather(mask_v, [ivec])
        offv = plsc.load_gather(off_v, [ivec])
        for l in range(_CHUNK // 16):
            raw16 = raw_v[i, pl.ds(c * _CHUNK + l * 16, 16)]
            base_v[j * _EMB + l, :] = ((raw16 & maskv) + offv) * _PADW
        return carry

    lax.fori_loop(0, _NCH, idx_body, 0)
    table_cp.wait()

    def tile_body(j, carry):
        i = j // _NC
        c = j - i * _NC
        tc = _NC * wid + c
        slot = (j % 4) * _EMB
        @pl.when(j >= 4)
        def _():
            pltpu.make_async_copy(
                tile_v.at[pl.ds(slot, _EMB)], out_hbm.at[i, tc], osem
            ).wait()
        for g in range(2):
            ls = [4 * g + k for k in range(4)]
            bases = [base_v[j * _EMB + l, :] for l in ls]
            for e in range(_EMB):
                vals = [plsc.load_gather(table_v, [b + e]) for b in bases]
                for l, v in zip(ls, vals):
                    tile_v[slot + e, pl.ds(l * 16, 16)] = v
        pltpu.async_copy(
            tile_v.at[pl.ds(slot, _EMB)], out_hbm.at[i, tc], osem)
        return carry

    lax.fori_loop(0, _NCH, tile_body, 0)
    for _ in range(4):
        pltpu.make_async_copy(
            tile_v.at[pl.ds(0, _EMB)], out_hbm.at[0, 0], osem
        ).wait()


def kernel(num_feats, cate_feats, W0, W1, W2, W3, W4, W5, W6, W7, W8):
    del num_feats
    flat_table = jnp.pad(
        jnp.concatenate([W0, W1, W2, W3, W4, W5, W6, W7, W8], axis=0),
        ((0, 0), (0, _PADW - _EMB))).reshape(_ROWS * _PADW)
    x = _emb_lookup(cate_feats.T, flat_table,
                    jnp.asarray(_MASK16), jnp.asarray(_OFF16))
    return x.transpose(1, 3, 0, 2).reshape(_B, _NT * _EMB)
